# baseline (speedup 1.0000x reference)
import jax
import jax.numpy as jnp
from jax.experimental import pallas as pl
from jax.experimental.pallas import tpu as pltpu
from jax.experimental.pallas import tpu_sc as plsc

_WINDOW = 128
_PLACE_NODES = 40


def _sc_gather_flat(data, idx_main, idx_tail, main, tail, d):
    total = main + tail

    @pl.kernel(
        out_type=jax.ShapeDtypeStruct((total, d), data.dtype),
        mesh=plsc.VectorSubcoreMesh(
            core_axis_name="core", subcore_axis_name="subcore"
        ),
    )
    def gather_kernel(data_hbm, idx_main_hbm, idx_tail_hbm, out_hbm):
        def body(idx_vmem, out_vmem):
            pltpu.sync_copy(data_hbm.at[idx_vmem.at[0]], out_vmem)

        pltpu.emit_pipeline(
            body,
            grid=(main // _WINDOW,),
            in_specs=[
                pl.BlockSpec((1, _WINDOW), index_map=lambda i: (0, i))
            ],
            out_specs=[
                pl.BlockSpec((_WINDOW, d), index_map=lambda i: (i, 0))
            ],
            core_axis_name=("core", "subcore"),
            dimension_semantics=(pltpu.PARALLEL,),
        )(idx_main_hbm, out_hbm)

        if tail:
            pltpu.emit_pipeline(
                body,
                grid=(1,),
                in_specs=[
                    pl.BlockSpec((1, tail), index_map=lambda i: (0, 0))
                ],
                out_specs=[
                    pl.BlockSpec(
                        (tail, d), index_map=lambda i: (main // tail, 0)
                    )
                ],
                core_axis_name=("core", "subcore"),
                dimension_semantics=(pltpu.PARALLEL,),
            )(idx_tail_hbm, out_hbm)

    return gather_kernel(data, idx_main, idx_tail)


def kernel(data, indices):
    n, d = data.shape
    k = indices.shape[1]
    total = n * (k + 1)
    neigh = indices.reshape(n, k).astype(jnp.int32)
    self_idx = jnp.arange(n, dtype=jnp.int32)[:, None]
    idx_flat = jnp.concatenate([self_idx, neigh], axis=1).reshape(total)
    main = (total // _WINDOW) * _WINDOW
    tail = total - main
    idx_main = idx_flat[:main].reshape(1, main)
    idx_tail = (
        idx_flat[main:].reshape(1, tail) if tail else idx_flat[:1].reshape(1, 1)
    )
    flat = _sc_gather_flat(data, idx_main, idx_tail, main, tail, d)
    one = data[0, 0] * 0.0 + 1.0
    return flat.reshape(n, k + 1, d) * one


# speedup vs baseline: 2.3956x; 2.3956x over previous
import jax
import jax.numpy as jnp
from jax.experimental import pallas as pl
from jax.experimental.pallas import tpu as pltpu
from jax.experimental.pallas import tpu_sc as plsc

_WINDOW = 128
_PLACE_NODES = 200


def _tc_place(flat, n, k1, d):
    nb = _PLACE_NODES

    def body(flat_ref, out_ref):
        out_ref[...] = flat_ref[...].reshape(nb, k1, d)

    return pl.pallas_call(
        body,
        grid=(n // nb,),
        in_specs=[
            pl.BlockSpec((nb * k1, d), lambda i: (i, 0)),
        ],
        out_specs=pl.BlockSpec((nb, k1, d), lambda i: (i, 0, 0)),
        out_shape=jax.ShapeDtypeStruct((n, k1, d), flat.dtype),
    )(flat)


def _sc_gather_flat(data, idx_main, idx_tail, main, tail, d):
    total = main + tail

    @pl.kernel(
        out_type=jax.ShapeDtypeStruct((total, d), data.dtype),
        mesh=plsc.VectorSubcoreMesh(
            core_axis_name="core", subcore_axis_name="subcore"
        ),
    )
    def gather_kernel(data_hbm, idx_main_hbm, idx_tail_hbm, out_hbm):
        def body(idx_vmem, out_vmem):
            pltpu.sync_copy(data_hbm.at[idx_vmem.at[0]], out_vmem)

        pltpu.emit_pipeline(
            body,
            grid=(main // _WINDOW,),
            in_specs=[
                pl.BlockSpec((1, _WINDOW), index_map=lambda i: (0, i))
            ],
            out_specs=[
                pl.BlockSpec((_WINDOW, d), index_map=lambda i: (i, 0))
            ],
            core_axis_name=("core", "subcore"),
            dimension_semantics=(pltpu.PARALLEL,),
        )(idx_main_hbm, out_hbm)

        if tail:
            pltpu.emit_pipeline(
                body,
                grid=(1,),
                in_specs=[
                    pl.BlockSpec((1, tail), index_map=lambda i: (0, 0))
                ],
                out_specs=[
                    pl.BlockSpec(
                        (tail, d), index_map=lambda i: (main // tail, 0)
                    )
                ],
                core_axis_name=("core", "subcore"),
                dimension_semantics=(pltpu.PARALLEL,),
            )(idx_tail_hbm, out_hbm)

    return gather_kernel(data, idx_main, idx_tail)


def kernel(data, indices):
    n, d = data.shape
    k = indices.shape[1]
    total = n * (k + 1)
    neigh = indices.reshape(n, k).astype(jnp.int32)
    self_idx = jnp.arange(n, dtype=jnp.int32)[:, None]
    idx_flat = jnp.concatenate([self_idx, neigh], axis=1).reshape(total)
    main = (total // _WINDOW) * _WINDOW
    tail = total - main
    idx_main = idx_flat[:main].reshape(1, main)
    idx_tail = (
        idx_flat[main:].reshape(1, tail) if tail else idx_flat[:1].reshape(1, 1)
    )
    flat = _sc_gather_flat(data, idx_main, idx_tail, main, tail, d)
    return _tc_place(flat, n, k + 1, d)


# speedup vs baseline: 2.4233x; 1.0115x over previous
import jax
import jax.numpy as jnp
from jax.experimental import pallas as pl
from jax.experimental.pallas import tpu as pltpu
from jax.experimental.pallas import tpu_sc as plsc

_WINDOW = 128
_PLACE_NODES = 400


def _tc_place(flat, n, k1, d):
    nb = _PLACE_NODES

    def body(flat_ref, out_ref):
        out_ref[...] = flat_ref[...].reshape(nb, k1, d)

    return pl.pallas_call(
        body,
        grid=(n // nb,),
        in_specs=[
            pl.BlockSpec((nb * k1, d), lambda i: (i, 0)),
        ],
        out_specs=pl.BlockSpec((nb, k1, d), lambda i: (i, 0, 0)),
        out_shape=jax.ShapeDtypeStruct((n, k1, d), flat.dtype),
    )(flat)


def _sc_gather_flat(data, idx_main, idx_tail, main, tail, d):
    total = main + tail

    @pl.kernel(
        out_type=jax.ShapeDtypeStruct((total, d), data.dtype),
        mesh=plsc.VectorSubcoreMesh(
            core_axis_name="core", subcore_axis_name="subcore"
        ),
    )
    def gather_kernel(data_hbm, idx_main_hbm, idx_tail_hbm, out_hbm):
        def body(idx_vmem, out_vmem):
            pltpu.sync_copy(data_hbm.at[idx_vmem.at[0]], out_vmem)

        pltpu.emit_pipeline(
            body,
            grid=(main // _WINDOW,),
            in_specs=[
                pl.BlockSpec((1, _WINDOW), index_map=lambda i: (0, i))
            ],
            out_specs=[
                pl.BlockSpec((_WINDOW, d), index_map=lambda i: (i, 0))
            ],
            core_axis_name=("core", "subcore"),
            dimension_semantics=(pltpu.PARALLEL,),
        )(idx_main_hbm, out_hbm)

        if tail:
            pltpu.emit_pipeline(
                body,
                grid=(1,),
                in_specs=[
                    pl.BlockSpec((1, tail), index_map=lambda i: (0, 0))
                ],
                out_specs=[
                    pl.BlockSpec(
                        (tail, d), index_map=lambda i: (main // tail, 0)
                    )
                ],
                core_axis_name=("core", "subcore"),
                dimension_semantics=(pltpu.PARALLEL,),
            )(idx_tail_hbm, out_hbm)

    return gather_kernel(data, idx_main, idx_tail)


def kernel(data, indices):
    n, d = data.shape
    k = indices.shape[1]
    total = n * (k + 1)
    neigh = indices.reshape(n, k).astype(jnp.int32)
    self_idx = jnp.arange(n, dtype=jnp.int32)[:, None]
    idx_flat = jnp.concatenate([self_idx, neigh], axis=1).reshape(total)
    main = (total // _WINDOW) * _WINDOW
    tail = total - main
    idx_main = idx_flat[:main].reshape(1, main)
    idx_tail = (
        idx_flat[main:].reshape(1, tail) if tail else idx_flat[:1].reshape(1, 1)
    )
    flat = _sc_gather_flat(data, idx_main, idx_tail, main, tail, d)
    return _tc_place(flat, n, k + 1, d)
